# baseline (speedup 1.0000x reference)
import functools

import jax
import jax.numpy as jnp
from jax import lax
from jax.experimental import pallas as pl
from jax.experimental.pallas import tpu as pltpu
from jax.experimental.pallas import tpu_sc as plsc

N = 10000
E = 320000
D_IN = 128
D_HID = 256
D_OUT = 64
K = 10

NCORE = 2
NTILE = 16
HALF = D_OUT // NCORE
CHUNK = 128
NSLOT = 6
LAG = 4
GROUP = NSLOT
TCHUNKS = 157
EPT = TCHUNKS * CHUNK
LROWS = 640
ZROWS = LROWS
NPAD = NTILE * LROWS
SENT = N


def _mlp_body(x_ref, w1_ref, b1_ref, w2_ref, b2_ref, o_ref):
    h = jnp.maximum(
        jnp.dot(x_ref[...], w1_ref[...], preferred_element_type=jnp.float32)
        + b1_ref[...], 0.0)
    h2 = (jnp.dot(h, w2_ref[...], preferred_element_type=jnp.float32)
          + b2_ref[...])
    o_ref[0] = h2[:, :HALF]
    o_ref[1] = h2[:, HALF:]


def _mlp(x, W1, b1, W2, b2):
    R = 1000
    grid = N // R
    return pl.pallas_call(
        _mlp_body,
        grid=(grid,),
        in_specs=[
            pl.BlockSpec((R, D_IN), lambda i: (i, 0)),
            pl.BlockSpec((D_IN, D_HID), lambda i: (0, 0)),
            pl.BlockSpec((1, D_HID), lambda i: (0, 0)),
            pl.BlockSpec((D_HID, D_OUT), lambda i: (0, 0)),
            pl.BlockSpec((1, D_OUT), lambda i: (0, 0)),
        ],
        out_specs=pl.BlockSpec((NCORE, R, HALF), lambda i: (0, i, 0)),
        out_shape=jax.ShapeDtypeStruct((NCORE, N, HALF), jnp.float32),
    )(x, W1, b1.reshape(1, D_HID), W2, b2.reshape(1, D_OUT))


def _prop_body(h0, srcr, dstr, tempb, out,
               srcbuf, dstbuf, stage, zbuf, zerob, tbuf,
               ha, hb, sem, gsem, ssem):
    cid = lax.axis_index("c")
    tid = lax.axis_index("s")

    pltpu.sync_copy(srcr.at[tid], srcbuf)
    pltpu.sync_copy(dstr.at[tid], dstbuf)
    pltpu.sync_copy(tempb, tbuf)

    pltpu.sync_copy(h0.at[cid, pl.ds(tid * LROWS, LROWS)],
                    ha.at[pl.ds(tid * LROWS, LROWS)])

    pltpu.sync_copy(h0.at[cid, pl.ds(tid * ZROWS, ZROWS)], zbuf)
    t0 = tbuf[0, :]

    def _zscale(r, _):
        zbuf[r, pl.ds(0, 16)] = zbuf[r, pl.ds(0, 16)] * t0
        zbuf[r, pl.ds(16, 16)] = zbuf[r, pl.ds(16, 16)] * t0
        return 0
    lax.fori_loop(0, ZROWS, _zscale, 0)

    zv = jnp.zeros((16,), jnp.float32)

    def _zzero(r, _):
        zerob[r, pl.ds(0, 16)] = zv
        zerob[r, pl.ds(16, 16)] = zv
        return 0
    lax.fori_loop(0, 64, _zzero, 0)

    def _issue_zeros(buf):
        return [pltpu.async_copy(
            zerob, buf.at[pl.ds(tid * LROWS + zc * 64, 64)], sem)
            for zc in range(LROWS // 64)]

    zcps = _issue_zeros(hb)

    for i in range(K):
        cur, nxt = (ha, hb) if i % 2 == 0 else (hb, ha)

        for cp in zcps:
            cp.wait()
        plsc.subcore_barrier()

        def _gather(c, s):
            return pltpu.async_copy(
                cur.at[srcbuf.at[c]], stage.at[s], gsem)

        def _scatter(c, s):
            return pltpu.async_copy(
                stage.at[s], nxt.at[dstbuf.at[c]], ssem, add=True)

        for s in range(LAG):
            _gather(s, s)
        for j in range(NSLOT - LAG):
            pltpu.make_async_copy(
                cur.at[srcbuf.at[j]], stage.at[j], gsem).wait()
            _scatter(j, j)
            _gather(j + LAG, (j + LAG) % NSLOT)

        def _pipe(j, _):
            s = lax.rem(j, NSLOT)
            s4 = lax.rem(j + LAG, NSLOT)
            pltpu.make_async_copy(
                cur.at[srcbuf.at[j]], stage.at[s], gsem).wait()
            _scatter(j, s)
            pltpu.make_async_copy(
                stage.at[s4], nxt.at[dstbuf.at[j - LAG]], ssem).wait()
            _gather(j + LAG, s4)
            return 0
        lax.fori_loop(NSLOT - LAG, TCHUNKS - LAG, _pipe, 0)

        def _tail(j, _):
            s = lax.rem(j, NSLOT)
            s4 = lax.rem(j + LAG, NSLOT)
            pltpu.make_async_copy(
                cur.at[srcbuf.at[j]], stage.at[s], gsem).wait()
            _scatter(j, s)
            pltpu.make_async_copy(
                stage.at[s4], nxt.at[dstbuf.at[j - LAG]], ssem).wait()
            return 0
        lax.fori_loop(TCHUNKS - LAG, TCHUNKS, _tail, 0)

        for s in range(NSLOT - LAG):
            pltpu.make_async_copy(
                stage.at[s], nxt.at[dstbuf.at[s]], ssem).wait()
        plsc.subcore_barrier()

        if i < K - 1:
            zcps = _issue_zeros(cur)

        tv = tbuf[i + 1, :]
        NZC = ZROWS // 128

        def _zrows(c):
            return nxt.at[pl.ds(tid * ZROWS + c * 128, 128)]

        pltpu.async_copy(_zrows(0), stage.at[0], gsem)
        for c5 in range(NZC):
            if c5 + 1 < NZC:
                pltpu.async_copy(_zrows(c5 + 1), stage.at[(c5 + 1) % 3],
                                 gsem)
            sl = stage.at[c5 % 3]
            pltpu.make_async_copy(_zrows(c5), sl, gsem).wait()

            def _zacc(r, _):
                row = c5 * 128 + r
                zbuf[row, pl.ds(0, 16)] = (
                    zbuf[row, pl.ds(0, 16)] + tv * sl[r, pl.ds(0, 16)])
                zbuf[row, pl.ds(16, 16)] = (
                    zbuf[row, pl.ds(16, 16)] + tv * sl[r, pl.ds(16, 16)])
                return 0
            lax.fori_loop(0, 128, _zacc, 0)

    pltpu.sync_copy(zbuf, out.at[cid, pl.ds(tid * ZROWS, ZROWS)])


def _propagate(h0p, srcr, dstr, tempb):
    mesh = plsc.VectorSubcoreMesh(core_axis_name="c", subcore_axis_name="s")
    return pl.kernel(
        _prop_body,
        out_type=jax.ShapeDtypeStruct((NCORE, NPAD, HALF), jnp.float32),
        mesh=mesh,
        compiler_params=pltpu.CompilerParams(use_tc_tiling_on_sc=False),
        scratch_types=[
            pltpu.VMEM((TCHUNKS, CHUNK), jnp.int32),
            pltpu.VMEM((TCHUNKS, CHUNK), jnp.int32),
            pltpu.VMEM((GROUP, CHUNK, HALF), jnp.float32),
            pltpu.VMEM((ZROWS, HALF), jnp.float32),
            pltpu.VMEM((64, HALF), jnp.float32),
            pltpu.VMEM((16, 16), jnp.float32),
            pltpu.VMEM_SHARED((NPAD, HALF), jnp.float32),
            pltpu.VMEM_SHARED((NPAD, HALF), jnp.float32),
            pltpu.SemaphoreType.DMA,
            pltpu.SemaphoreType.DMA,
            pltpu.SemaphoreType.DMA,
        ],
    )(h0p, srcr, dstr, tempb)


@jax.jit
def kernel(x, edge_index, W1, b1, W2, b2, temp):
    h0 = _mlp(x, W1, b1, W2, b2)
    h0p = jnp.pad(h0, ((0, 0), (0, NPAD - N), (0, 0)))

    dst = edge_index[0]
    src = edge_index[1]
    rpt = E // NTILE
    pad = ((0, 0), (0, EPT - rpt))
    srcr = jnp.pad(src.reshape(NTILE, rpt), pad,
                   constant_values=SENT).reshape(NTILE, TCHUNKS, CHUNK)
    dstr = jnp.pad(dst.reshape(NTILE, rpt), pad,
                   constant_values=SENT).reshape(NTILE, TCHUNKS, CHUNK)

    tpad = jnp.pad(temp, (0, 16 - (K + 1)))
    tempb = jnp.broadcast_to(tpad[:, None], (16, 16))

    z = _propagate(h0p, srcr, dstr, tempb)
    return z[:, :N].transpose(1, 0, 2).reshape(N, D_OUT)


# speedup vs baseline: 1.0176x; 1.0176x over previous
import functools

import jax
import jax.numpy as jnp
from jax import lax
from jax.experimental import pallas as pl
from jax.experimental.pallas import tpu as pltpu
from jax.experimental.pallas import tpu_sc as plsc

N = 10000
E = 320000
D_IN = 128
D_HID = 256
D_OUT = 64
K = 10

NCORE = 2
NTILE = 16
HALF = D_OUT // NCORE
CHUNK = 128
NSLOT = 6
LAG = 4
GROUP = NSLOT
TCHUNKS = 157
EPT = TCHUNKS * CHUNK
LROWS = 640
ZROWS = LROWS
NPAD = NTILE * LROWS
SENT = N


def _mlp_body(x_ref, w1_ref, b1_ref, w2_ref, b2_ref, o_ref):
    h = jnp.maximum(
        jnp.dot(x_ref[...], w1_ref[...], preferred_element_type=jnp.float32)
        + b1_ref[...], 0.0)
    h2 = (jnp.dot(h, w2_ref[...], preferred_element_type=jnp.float32)
          + b2_ref[...])
    o_ref[0] = h2[:, :HALF]
    o_ref[1] = h2[:, HALF:]


def _mlp(x, W1, b1, W2, b2):
    R = 1000
    grid = N // R
    return pl.pallas_call(
        _mlp_body,
        grid=(grid,),
        in_specs=[
            pl.BlockSpec((R, D_IN), lambda i: (i, 0)),
            pl.BlockSpec((D_IN, D_HID), lambda i: (0, 0)),
            pl.BlockSpec((1, D_HID), lambda i: (0, 0)),
            pl.BlockSpec((D_HID, D_OUT), lambda i: (0, 0)),
            pl.BlockSpec((1, D_OUT), lambda i: (0, 0)),
        ],
        out_specs=pl.BlockSpec((NCORE, R, HALF), lambda i: (0, i, 0)),
        out_shape=jax.ShapeDtypeStruct((NCORE, N, HALF), jnp.float32),
    )(x, W1, b1.reshape(1, D_HID), W2, b2.reshape(1, D_OUT))


def _prop_body(h0, srcr, dstr, tempb, out,
               srcbuf, dstbuf, stage, zbuf, zerob, tbuf,
               ha, hb, sem, gsem, ssem):
    cid = lax.axis_index("c")
    tid = lax.axis_index("s")

    pltpu.sync_copy(srcr.at[tid], srcbuf)
    pltpu.sync_copy(dstr.at[tid], dstbuf)
    pltpu.sync_copy(tempb, tbuf)

    pltpu.sync_copy(h0.at[cid, pl.ds(tid * LROWS, LROWS)],
                    ha.at[pl.ds(tid * LROWS, LROWS)])

    pltpu.sync_copy(h0.at[cid, pl.ds(tid * ZROWS, ZROWS)], zbuf)
    t0 = tbuf[0, :]

    def _zscale(r, _):
        zbuf[r, pl.ds(0, 16)] = zbuf[r, pl.ds(0, 16)] * t0
        zbuf[r, pl.ds(16, 16)] = zbuf[r, pl.ds(16, 16)] * t0
        return 0
    lax.fori_loop(0, ZROWS, _zscale, 0)

    zv = jnp.zeros((16,), jnp.float32)

    def _zzero(r, _):
        zerob[r, pl.ds(0, 16)] = zv
        zerob[r, pl.ds(16, 16)] = zv
        return 0
    lax.fori_loop(0, 64, _zzero, 0)

    def _issue_zeros(buf):
        return [pltpu.async_copy(
            zerob, buf.at[pl.ds(tid * LROWS + zc * 64, 64)], sem)
            for zc in range(LROWS // 64)]

    zcps = _issue_zeros(hb)

    for i in range(K):
        cur, nxt = (ha, hb) if i % 2 == 0 else (hb, ha)

        for cp in zcps:
            cp.wait()
        plsc.subcore_barrier()

        def _gather(c, s):
            return pltpu.async_copy(
                cur.at[srcbuf.at[c]], stage.at[s], gsem)

        def _scatter(c, s):
            return pltpu.async_copy(
                stage.at[s], nxt.at[dstbuf.at[c]], ssem, add=True)

        for s in range(LAG):
            _gather(s, s)
        for j in range(NSLOT - LAG):
            pltpu.make_async_copy(
                cur.at[srcbuf.at[j]], stage.at[j], gsem).wait()
            _scatter(j, j)
            _gather(j + LAG, (j + LAG) % NSLOT)

        def _pipe(j, _):
            s = lax.rem(j, NSLOT)
            s4 = lax.rem(j + LAG, NSLOT)
            pltpu.make_async_copy(
                cur.at[srcbuf.at[j]], stage.at[s], gsem).wait()
            _scatter(j, s)
            pltpu.make_async_copy(
                stage.at[s4], nxt.at[dstbuf.at[j - LAG]], ssem).wait()
            _gather(j + LAG, s4)
            return 0
        lax.fori_loop(NSLOT - LAG, TCHUNKS - LAG, _pipe, 0, unroll=2)

        def _tail(j, _):
            s = lax.rem(j, NSLOT)
            s4 = lax.rem(j + LAG, NSLOT)
            pltpu.make_async_copy(
                cur.at[srcbuf.at[j]], stage.at[s], gsem).wait()
            _scatter(j, s)
            pltpu.make_async_copy(
                stage.at[s4], nxt.at[dstbuf.at[j - LAG]], ssem).wait()
            return 0
        lax.fori_loop(TCHUNKS - LAG, TCHUNKS, _tail, 0)

        for s in range(NSLOT - LAG):
            pltpu.make_async_copy(
                stage.at[s], nxt.at[dstbuf.at[s]], ssem).wait()
        plsc.subcore_barrier()

        if i < K - 1:
            zcps = _issue_zeros(cur)

        tv = tbuf[i + 1, :]
        NZC = ZROWS // 128

        def _zrows(c):
            return nxt.at[pl.ds(tid * ZROWS + c * 128, 128)]

        pltpu.async_copy(_zrows(0), stage.at[0], gsem)
        for c5 in range(NZC):
            if c5 + 1 < NZC:
                pltpu.async_copy(_zrows(c5 + 1), stage.at[(c5 + 1) % 3],
                                 gsem)
            sl = stage.at[c5 % 3]
            pltpu.make_async_copy(_zrows(c5), sl, gsem).wait()

            def _zacc(r4, _):
                for rr in range(4):
                    row = c5 * 128 + r4 * 4 + rr
                    sr = r4 * 4 + rr
                    zbuf[row, pl.ds(0, 16)] = (
                        zbuf[row, pl.ds(0, 16)]
                        + tv * sl[sr, pl.ds(0, 16)])
                    zbuf[row, pl.ds(16, 16)] = (
                        zbuf[row, pl.ds(16, 16)]
                        + tv * sl[sr, pl.ds(16, 16)])
                return 0
            lax.fori_loop(0, 32, _zacc, 0)

    pltpu.sync_copy(zbuf, out.at[cid, pl.ds(tid * ZROWS, ZROWS)])


def _propagate(h0p, srcr, dstr, tempb):
    mesh = plsc.VectorSubcoreMesh(core_axis_name="c", subcore_axis_name="s")
    return pl.kernel(
        _prop_body,
        out_type=jax.ShapeDtypeStruct((NCORE, NPAD, HALF), jnp.float32),
        mesh=mesh,
        compiler_params=pltpu.CompilerParams(use_tc_tiling_on_sc=False),
        scratch_types=[
            pltpu.VMEM((TCHUNKS, CHUNK), jnp.int32),
            pltpu.VMEM((TCHUNKS, CHUNK), jnp.int32),
            pltpu.VMEM((GROUP, CHUNK, HALF), jnp.float32),
            pltpu.VMEM((ZROWS, HALF), jnp.float32),
            pltpu.VMEM((64, HALF), jnp.float32),
            pltpu.VMEM((16, 16), jnp.float32),
            pltpu.VMEM_SHARED((NPAD, HALF), jnp.float32),
            pltpu.VMEM_SHARED((NPAD, HALF), jnp.float32),
            pltpu.SemaphoreType.DMA,
            pltpu.SemaphoreType.DMA,
            pltpu.SemaphoreType.DMA,
        ],
    )(h0p, srcr, dstr, tempb)


@jax.jit
def kernel(x, edge_index, W1, b1, W2, b2, temp):
    h0 = _mlp(x, W1, b1, W2, b2)
    h0p = jnp.pad(h0, ((0, 0), (0, NPAD - N), (0, 0)))

    dst = edge_index[0]
    src = edge_index[1]
    rpt = E // NTILE
    pad = ((0, 0), (0, EPT - rpt))
    srcr = jnp.pad(src.reshape(NTILE, rpt), pad,
                   constant_values=SENT).reshape(NTILE, TCHUNKS, CHUNK)
    dstr = jnp.pad(dst.reshape(NTILE, rpt), pad,
                   constant_values=SENT).reshape(NTILE, TCHUNKS, CHUNK)

    tpad = jnp.pad(temp, (0, 16 - (K + 1)))
    tempb = jnp.broadcast_to(tpad[:, None], (16, 16))

    z = _propagate(h0p, srcr, dstr, tempb)
    return z[:, :N].transpose(1, 0, 2).reshape(N, D_OUT)
